# Initial kernel scaffold; baseline (speedup 1.0000x reference)
import jax
import jax.numpy as jnp
from jax.experimental import pallas as pl


def kernel(node_feats, edge_feats, edge_index, edge_indices, W1, b1, W2, b2, conv_bias, cW1, cb1, cW2, cb2):
    raise NotImplementedError("write your pallas kernel here")


# speedup vs baseline: 2.8571x; 2.8571x over previous
import functools

import jax
import jax.numpy as jnp
from jax import lax
from jax.experimental import pallas as pl
from jax.experimental.pallas import tpu as pltpu
from jax.experimental.pallas import tpu_sc as plsc

N_NODES = 10000
E_EDGES = 160000
IN_F = 128
DE_F = 16
H_F = 4
OUT_F = 2
MID_F = 256
K_SEL = 100000
KP = 102400

NC = 2
NS = 16
NW = NC * NS

EPT = E_EDGES // NW
EFULL = EPT // 128
ETAIL = EPT - EFULL * 128
NPAD = 10240
NPT = NPAD // NS

@functools.cache
def _mesh():
    return plsc.VectorSubcoreMesh(core_axis_name="c", subcore_axis_name="s")


def _wid():
    return lax.axis_index("s") * NC + lax.axis_index("c")


def _sc_gather_xsrc(table, idx):
    return pl.kernel(
        _sc_gather_xsrc_body,
        out_type=jax.ShapeDtypeStruct((E_EDGES, IN_F), jnp.float32),
        mesh=_mesh(),
        scratch_types=[
            pltpu.VMEM((128,), jnp.int32),
            pltpu.VMEM((128, IN_F), jnp.float32),
            pltpu.VMEM((ETAIL,), jnp.int32),
            pltpu.VMEM((ETAIL, IN_F), jnp.float32),
            pltpu.SemaphoreType.DMA,
        ],
    )(table, idx)


def _sc_gather_xsrc_body(table, idx, out, idx_v, rows_v, idx_t, rows_t, sem):
    base = _wid() * EPT

    def step(j, carry):
        off = base + j * 128
        pltpu.sync_copy(idx.at[pl.ds(off, 128)], idx_v)
        pltpu.async_copy(table.at[idx_v], rows_v, sem).wait()
        pltpu.sync_copy(rows_v, out.at[pl.ds(off, 128)])
        return carry

    lax.fori_loop(0, EFULL, step, 0)
    off = base + EFULL * 128
    pltpu.sync_copy(idx.at[pl.ds(off, ETAIL)], idx_t)
    pltpu.async_copy(table.at[idx_t], rows_t, sem).wait()
    pltpu.sync_copy(rows_t, out.at[pl.ds(off, ETAIL)])


def _tc_msg_body(ef_ref, xs_ref, w1_ref, b1_ref, w2p_ref, b2p_ref, out_ref):
    he = jnp.maximum(
        jnp.dot(ef_ref[...], w1_ref[...], preferred_element_type=jnp.float32)
        + b1_ref[...],
        0.0,
    )
    we2 = (
        jnp.dot(he, w2p_ref[...], preferred_element_type=jnp.float32)
        + b2p_ref[...]
    )
    xs = xs_ref[...]
    cols = [
        jnp.sum(xs * we2[:, o * IN_F:(o + 1) * IN_F], axis=1, keepdims=True)
        for o in range(H_F)
    ]
    ones = jnp.ones_like(cols[0])
    zeros = jnp.zeros((xs.shape[0], 8 - H_F - 1), jnp.float32)
    out_ref[...] = jnp.concatenate(cols + [ones, zeros], axis=1)


def _tc_msg(edge_feats, x_src, W1, b1, W2p, b2p):
    tile = 640
    grid = E_EDGES // tile
    return pl.pallas_call(
        _tc_msg_body,
        grid=(grid,),
        in_specs=[
            pl.BlockSpec((tile, DE_F), lambda i: (i, 0)),
            pl.BlockSpec((tile, IN_F), lambda i: (i, 0)),
            pl.BlockSpec((DE_F, MID_F), lambda i: (0, 0)),
            pl.BlockSpec((1, MID_F), lambda i: (0, 0)),
            pl.BlockSpec((MID_F, IN_F * H_F), lambda i: (0, 0)),
            pl.BlockSpec((1, IN_F * H_F), lambda i: (0, 0)),
        ],
        out_specs=pl.BlockSpec((tile, 8), lambda i: (i, 0)),
        out_shape=jax.ShapeDtypeStruct((E_EDGES, 8), jnp.float32),
    )(edge_feats, x_src, W1, b1.reshape(1, -1), W2p, b2p.reshape(1, -1))


def _sc_scatter_msg(msgp, dstidx, zrows):
    return pl.kernel(
        _sc_scatter_msg_body,
        out_type=jax.ShapeDtypeStruct((NC, NPAD, 8), jnp.float32),
        mesh=_mesh(),
        scratch_types=[
            pltpu.VMEM((128,), jnp.int32),
            pltpu.VMEM((128, 8), jnp.float32),
            pltpu.VMEM((ETAIL,), jnp.int32),
            pltpu.VMEM((ETAIL, 8), jnp.float32),
            pltpu.VMEM_SHARED((NPAD, 8), jnp.float32),
        ],
        compiler_params=pltpu.CompilerParams(use_tc_tiling_on_sc=False),
    )(msgp, dstidx, zrows)


def _sc_scatter_msg_body(msgp, dstidx, zrows, out, idx_v, msg_v, idx_t, msg_t, acc):
    cid = lax.axis_index("c")
    sid = lax.axis_index("s")
    base = _wid() * EPT
    nb = sid * NPT
    pltpu.sync_copy(zrows.at[pl.ds(nb, NPT)], acc.at[pl.ds(nb, NPT)])
    plsc.subcore_barrier()

    def step(j, carry):
        off = base + j * 128
        pltpu.sync_copy(dstidx.at[pl.ds(off, 128)], idx_v)
        pltpu.sync_copy(msgp.at[pl.ds(off, 128)], msg_v)
        pltpu.sync_copy(msg_v, acc.at[idx_v], add=True)
        return carry

    lax.fori_loop(0, EFULL, step, 0)
    off = base + EFULL * 128
    pltpu.sync_copy(dstidx.at[pl.ds(off, ETAIL)], idx_t)
    pltpu.sync_copy(msgp.at[pl.ds(off, ETAIL)], msg_t)
    pltpu.sync_copy(msg_t, acc.at[idx_t], add=True)
    plsc.subcore_barrier()
    pltpu.sync_copy(acc.at[pl.ds(nb, NPT)], out.at[cid, pl.ds(nb, NPT)])


def _tc_finish_body(p_ref, bias_ref, sel_ref, out_ref):
    s = p_ref[0] + p_ref[1]
    deg = jnp.sum(s * sel_ref[...], axis=1, keepdims=True)
    h = jnp.maximum(s / jnp.maximum(deg, 1.0) + bias_ref[...], 0.0)
    mask = jnp.concatenate(
        [jnp.ones((1, H_F), jnp.float32), jnp.zeros((1, 8 - H_F), jnp.float32)],
        axis=1,
    )
    h = h * mask
    out_ref[...] = jnp.concatenate([h, jnp.zeros_like(h)], axis=1)


def _tc_finish(partials, conv_bias):
    bias8 = jnp.pad(conv_bias, (0, 8 - H_F)).reshape(1, 8)
    sel = jnp.zeros((1, 8), jnp.float32).at[0, H_F].set(1.0)
    return pl.pallas_call(
        _tc_finish_body,
        grid=(1,),
        in_specs=[
            pl.BlockSpec((NC, NPAD, 8), lambda i: (0, 0, 0)),
            pl.BlockSpec((1, 8), lambda i: (0, 0)),
            pl.BlockSpec((1, 8), lambda i: (0, 0)),
        ],
        out_specs=pl.BlockSpec((NPAD, 16), lambda i: (0, 0)),
        out_shape=jax.ShapeDtypeStruct((NPAD, 16), jnp.float32),
    )(partials, bias8, sel)


_CPT = KP // 128 // NW


def _sc_cls_gather(eidx, srcids, dstids, h16, efeat):
    return pl.kernel(
        _sc_cls_gather_body,
        out_type=(
            jax.ShapeDtypeStruct((KP, 16), jnp.float32),
            jax.ShapeDtypeStruct((KP, 16), jnp.float32),
            jax.ShapeDtypeStruct((KP, DE_F), jnp.float32),
        ),
        mesh=_mesh(),
        scratch_types=[
            pltpu.VMEM((128,), jnp.int32),
            pltpu.VMEM((128,), jnp.int32),
            pltpu.VMEM((128,), jnp.int32),
            pltpu.VMEM((128, 16), jnp.float32),
            pltpu.VMEM((128, 16), jnp.float32),
            pltpu.VMEM((128, DE_F), jnp.float32),
            pltpu.SemaphoreType.DMA,
        ],
        compiler_params=pltpu.CompilerParams(use_tc_tiling_on_sc=False),
    )(eidx, srcids, dstids, h16, efeat)


def _sc_cls_gather_body(eidx, srcids, dstids, h16, efeat, o_s, o_d, o_e,
                        eidx_v, src_v, dst_v, hs_v, hd_v, ef_v, sem):
    base0 = _wid() * _CPT * 128

    def step(j, carry):
        off = base0 + j * 128
        pltpu.sync_copy(eidx.at[pl.ds(off, 128)], eidx_v)
        pltpu.async_copy(srcids.at[eidx_v], src_v, sem).wait()
        pltpu.async_copy(dstids.at[eidx_v], dst_v, sem).wait()
        pltpu.async_copy(h16.at[src_v], hs_v, sem).wait()
        pltpu.async_copy(h16.at[dst_v], hd_v, sem).wait()
        pltpu.async_copy(efeat.at[eidx_v], ef_v, sem).wait()
        pltpu.sync_copy(hs_v, o_s.at[pl.ds(off, 128)])
        pltpu.sync_copy(hd_v, o_d.at[pl.ds(off, 128)])
        pltpu.sync_copy(ef_v, o_e.at[pl.ds(off, 128)])
        return carry

    lax.fori_loop(0, _CPT, step, 0)


def _tc_cls_body(a_ref, b_ref, c_ref, w1_ref, b1_ref, w2_ref, b2_ref, out_ref):
    ei = jnp.concatenate([a_ref[...], b_ref[...], c_ref[...]], axis=1)
    z = jnp.maximum(
        jnp.dot(ei, w1_ref[...], preferred_element_type=jnp.float32)
        + b1_ref[...],
        0.0,
    )
    out_ref[...] = (
        jnp.dot(z, w2_ref[...], preferred_element_type=jnp.float32)
        + b2_ref[...]
    )


def _tc_cls(o_s, o_d, o_e, cW1, cb1, cW2, cb2):
    blk1 = jnp.pad(cW1[0:H_F], ((0, 16 - H_F), (0, 8 - H_F)))
    blk2 = jnp.pad(cW1[H_F:2 * H_F], ((0, 16 - H_F), (0, 8 - H_F)))
    blk3 = jnp.pad(cW1[2 * H_F:], ((0, 0), (0, 8 - H_F)))
    cW1p = jnp.concatenate([blk1, blk2, blk3], axis=0)
    cb1p = jnp.pad(cb1, (0, 8 - H_F)).reshape(1, 8)
    cW2p = jnp.pad(cW2, ((0, 8 - H_F), (0, 8 - OUT_F)))
    cb2p = jnp.pad(cb2, (0, 8 - OUT_F)).reshape(1, 8)
    tile = 1024
    grid = KP // tile
    return pl.pallas_call(
        _tc_cls_body,
        grid=(grid,),
        in_specs=[
            pl.BlockSpec((tile, 16), lambda i: (i, 0)),
            pl.BlockSpec((tile, 16), lambda i: (i, 0)),
            pl.BlockSpec((tile, DE_F), lambda i: (i, 0)),
            pl.BlockSpec((32 + DE_F, 8), lambda i: (0, 0)),
            pl.BlockSpec((1, 8), lambda i: (0, 0)),
            pl.BlockSpec((8, 8), lambda i: (0, 0)),
            pl.BlockSpec((1, 8), lambda i: (0, 0)),
        ],
        out_specs=pl.BlockSpec((tile, 8), lambda i: (i, 0)),
        out_shape=jax.ShapeDtypeStruct((KP, 8), jnp.float32),
    )(o_s, o_d, o_e, cW1p, cb1p, cW2p, cb2p)


def kernel(node_feats, edge_feats, edge_index, edge_indices,
           W1, b1, W2, b2, conv_bias, cW1, cb1, cW2, cb2):
    src = edge_index[0]
    dst = edge_index[1]
    W2p = W2.reshape(MID_F, IN_F, H_F).transpose(0, 2, 1).reshape(MID_F, IN_F * H_F)
    b2p = b2.reshape(IN_F, H_F).T.reshape(IN_F * H_F)

    x_src = _sc_gather_xsrc(node_feats, src)
    msgp = _tc_msg(edge_feats, x_src, W1, b1, W2p, b2p)
    zrows = jnp.zeros((NPAD, 8), jnp.float32)
    partials = _sc_scatter_msg(msgp, dst, zrows)
    h16 = _tc_finish(partials, conv_bias)

    eidxp = jnp.concatenate(
        [edge_indices, jnp.zeros((KP - K_SEL,), jnp.int32)])
    o_s, o_d, o_e = _sc_cls_gather(eidxp, src, dst, h16, edge_feats)
    out8 = _tc_cls(o_s, o_d, o_e, cW1, cb1, cW2, cb2)
    return out8[:K_SEL, :OUT_F]


# speedup vs baseline: 2.8967x; 1.0138x over previous
import functools

import jax
import jax.numpy as jnp
from jax import lax
from jax.experimental import pallas as pl
from jax.experimental.pallas import tpu as pltpu
from jax.experimental.pallas import tpu_sc as plsc

N_NODES = 10000
E_EDGES = 160000
IN_F = 128
DE_F = 16
H_F = 4
OUT_F = 2
MID_F = 256
K_SEL = 100000
KP = 102400

NC = 2
NS = 16
NW = NC * NS

EPT = E_EDGES // NW
EFULL = EPT // 128
ETAIL = EPT - EFULL * 128
NPAD = 10240
NPT = NPAD // NS

@functools.cache
def _mesh():
    return plsc.VectorSubcoreMesh(core_axis_name="c", subcore_axis_name="s")


def _wid():
    return lax.axis_index("s") * NC + lax.axis_index("c")


def _sc_gather_xsrc(table, idx):
    return pl.kernel(
        _sc_gather_xsrc_body,
        out_type=jax.ShapeDtypeStruct((E_EDGES, IN_F), jnp.float32),
        mesh=_mesh(),
        scratch_types=[
            pltpu.VMEM((128,), jnp.int32),
            pltpu.VMEM((128, IN_F), jnp.float32),
            pltpu.VMEM((ETAIL,), jnp.int32),
            pltpu.VMEM((ETAIL, IN_F), jnp.float32),
            pltpu.SemaphoreType.DMA,
        ],
    )(table, idx)


def _sc_gather_xsrc_body(table, idx, out, idx_v, rows_v, idx_t, rows_t, sem):
    base = _wid() * EPT

    def step(j, carry):
        off = base + j * 128
        pltpu.sync_copy(idx.at[pl.ds(off, 128)], idx_v)
        pltpu.async_copy(table.at[idx_v], rows_v, sem).wait()
        pltpu.sync_copy(rows_v, out.at[pl.ds(off, 128)])
        return carry

    lax.fori_loop(0, EFULL, step, 0)
    off = base + EFULL * 128
    pltpu.sync_copy(idx.at[pl.ds(off, ETAIL)], idx_t)
    pltpu.async_copy(table.at[idx_t], rows_t, sem).wait()
    pltpu.sync_copy(rows_t, out.at[pl.ds(off, ETAIL)])


def _tc_msg_body(ef_ref, xs_ref, w1_ref, b1_ref, w2p_ref, b2p_ref, out_ref):
    he = jnp.maximum(
        jnp.dot(ef_ref[...], w1_ref[...], preferred_element_type=jnp.float32)
        + b1_ref[...],
        0.0,
    )
    we2 = (
        jnp.dot(he.astype(jnp.bfloat16), w2p_ref[...],
                preferred_element_type=jnp.float32)
        + b2p_ref[...]
    )
    xs = xs_ref[...]
    cols = [
        jnp.sum(xs * we2[:, o * IN_F:(o + 1) * IN_F], axis=1, keepdims=True)
        for o in range(H_F)
    ]
    ones = jnp.ones_like(cols[0])
    zeros = jnp.zeros((xs.shape[0], 8 - H_F - 1), jnp.float32)
    out_ref[...] = jnp.concatenate(cols + [ones, zeros], axis=1)


def _tc_msg(edge_feats, x_src, W1, b1, W2p, b2p):
    tile = 640
    grid = E_EDGES // tile
    return pl.pallas_call(
        _tc_msg_body,
        grid=(grid,),
        in_specs=[
            pl.BlockSpec((tile, DE_F), lambda i: (i, 0)),
            pl.BlockSpec((tile, IN_F), lambda i: (i, 0)),
            pl.BlockSpec((DE_F, MID_F), lambda i: (0, 0)),
            pl.BlockSpec((1, MID_F), lambda i: (0, 0)),
            pl.BlockSpec((MID_F, IN_F * H_F), lambda i: (0, 0)),
            pl.BlockSpec((1, IN_F * H_F), lambda i: (0, 0)),
        ],
        out_specs=pl.BlockSpec((tile, 8), lambda i: (i, 0)),
        out_shape=jax.ShapeDtypeStruct((E_EDGES, 8), jnp.float32),
    )(edge_feats, x_src, W1, b1.reshape(1, -1),
      W2p.astype(jnp.bfloat16), b2p.reshape(1, -1))


def _sc_scatter_msg(msgp, dstidx, zrows):
    return pl.kernel(
        _sc_scatter_msg_body,
        out_type=jax.ShapeDtypeStruct((NC, NPAD, 8), jnp.float32),
        mesh=_mesh(),
        scratch_types=[
            pltpu.VMEM((128,), jnp.int32),
            pltpu.VMEM((128, 8), jnp.float32),
            pltpu.VMEM((ETAIL,), jnp.int32),
            pltpu.VMEM((ETAIL, 8), jnp.float32),
            pltpu.VMEM_SHARED((NPAD, 8), jnp.float32),
        ],
        compiler_params=pltpu.CompilerParams(use_tc_tiling_on_sc=False),
    )(msgp, dstidx, zrows)


def _sc_scatter_msg_body(msgp, dstidx, zrows, out, idx_v, msg_v, idx_t, msg_t, acc):
    cid = lax.axis_index("c")
    sid = lax.axis_index("s")
    base = _wid() * EPT
    nb = sid * NPT
    pltpu.sync_copy(zrows.at[pl.ds(nb, NPT)], acc.at[pl.ds(nb, NPT)])
    plsc.subcore_barrier()

    def step(j, carry):
        off = base + j * 128
        pltpu.sync_copy(dstidx.at[pl.ds(off, 128)], idx_v)
        pltpu.sync_copy(msgp.at[pl.ds(off, 128)], msg_v)
        pltpu.sync_copy(msg_v, acc.at[idx_v], add=True)
        return carry

    lax.fori_loop(0, EFULL, step, 0)
    off = base + EFULL * 128
    pltpu.sync_copy(dstidx.at[pl.ds(off, ETAIL)], idx_t)
    pltpu.sync_copy(msgp.at[pl.ds(off, ETAIL)], msg_t)
    pltpu.sync_copy(msg_t, acc.at[idx_t], add=True)
    plsc.subcore_barrier()
    pltpu.sync_copy(acc.at[pl.ds(nb, NPT)], out.at[cid, pl.ds(nb, NPT)])


def _tc_finish_body(p_ref, bias_ref, sel_ref, out_ref):
    s = p_ref[0] + p_ref[1]
    deg = jnp.sum(s * sel_ref[...], axis=1, keepdims=True)
    h = jnp.maximum(s / jnp.maximum(deg, 1.0) + bias_ref[...], 0.0)
    mask = jnp.concatenate(
        [jnp.ones((1, H_F), jnp.float32), jnp.zeros((1, 8 - H_F), jnp.float32)],
        axis=1,
    )
    h = h * mask
    out_ref[...] = jnp.concatenate([h, jnp.zeros_like(h)], axis=1)


def _tc_finish(partials, conv_bias):
    bias8 = jnp.pad(conv_bias, (0, 8 - H_F)).reshape(1, 8)
    sel = jnp.zeros((1, 8), jnp.float32).at[0, H_F].set(1.0)
    return pl.pallas_call(
        _tc_finish_body,
        grid=(1,),
        in_specs=[
            pl.BlockSpec((NC, NPAD, 8), lambda i: (0, 0, 0)),
            pl.BlockSpec((1, 8), lambda i: (0, 0)),
            pl.BlockSpec((1, 8), lambda i: (0, 0)),
        ],
        out_specs=pl.BlockSpec((NPAD, 16), lambda i: (0, 0)),
        out_shape=jax.ShapeDtypeStruct((NPAD, 16), jnp.float32),
    )(partials, bias8, sel)


_CPT = KP // 128 // NW


def _sc_cls_gather_ids(eidx, srcids, dstids, efeat):
    return pl.kernel(
        _sc_cls_gather_ids_body,
        out_type=(
            jax.ShapeDtypeStruct((KP,), jnp.int32),
            jax.ShapeDtypeStruct((KP,), jnp.int32),
            jax.ShapeDtypeStruct((KP, DE_F), jnp.float32),
        ),
        mesh=_mesh(),
        scratch_types=[
            pltpu.VMEM((128,), jnp.int32),
            pltpu.VMEM((128,), jnp.int32),
            pltpu.VMEM((128,), jnp.int32),
            pltpu.VMEM((128, DE_F), jnp.float32),
            pltpu.SemaphoreType.DMA,
        ],
        compiler_params=pltpu.CompilerParams(use_tc_tiling_on_sc=False),
    )(eidx, srcids, dstids, efeat)


def _sc_cls_gather_ids_body(eidx, srcids, dstids, efeat, o_es, o_ed, o_e,
                            eidx_v, src_v, dst_v, ef_v, sem):
    base0 = _wid() * _CPT * 128

    def step(j, carry):
        off = base0 + j * 128
        pltpu.sync_copy(eidx.at[pl.ds(off, 128)], eidx_v)
        pltpu.async_copy(srcids.at[eidx_v], src_v, sem).wait()
        pltpu.async_copy(dstids.at[eidx_v], dst_v, sem).wait()
        pltpu.async_copy(efeat.at[eidx_v], ef_v, sem).wait()
        pltpu.sync_copy(src_v, o_es.at[pl.ds(off, 128)])
        pltpu.sync_copy(dst_v, o_ed.at[pl.ds(off, 128)])
        pltpu.sync_copy(ef_v, o_e.at[pl.ds(off, 128)])
        return carry

    lax.fori_loop(0, _CPT, step, 0)


def _sc_cls_gather_h(es, ed, h16):
    return pl.kernel(
        _sc_cls_gather_h_body,
        out_type=(
            jax.ShapeDtypeStruct((KP, 16), jnp.float32),
            jax.ShapeDtypeStruct((KP, 16), jnp.float32),
        ),
        mesh=_mesh(),
        scratch_types=[
            pltpu.VMEM((128,), jnp.int32),
            pltpu.VMEM((128,), jnp.int32),
            pltpu.VMEM((128, 16), jnp.float32),
            pltpu.VMEM((128, 16), jnp.float32),
            pltpu.SemaphoreType.DMA,
        ],
        compiler_params=pltpu.CompilerParams(use_tc_tiling_on_sc=False),
    )(es, ed, h16)


def _sc_cls_gather_h_body(es, ed, h16, o_s, o_d,
                          src_v, dst_v, hs_v, hd_v, sem):
    base0 = _wid() * _CPT * 128

    def step(j, carry):
        off = base0 + j * 128
        pltpu.sync_copy(es.at[pl.ds(off, 128)], src_v)
        pltpu.sync_copy(ed.at[pl.ds(off, 128)], dst_v)
        pltpu.async_copy(h16.at[src_v], hs_v, sem).wait()
        pltpu.async_copy(h16.at[dst_v], hd_v, sem).wait()
        pltpu.sync_copy(hs_v, o_s.at[pl.ds(off, 128)])
        pltpu.sync_copy(hd_v, o_d.at[pl.ds(off, 128)])
        return carry

    lax.fori_loop(0, _CPT, step, 0)


def _tc_cls_body(a_ref, b_ref, c_ref, w1_ref, b1_ref, w2_ref, b2_ref, out_ref):
    ei = jnp.concatenate([a_ref[...], b_ref[...], c_ref[...]], axis=1)
    z = jnp.maximum(
        jnp.dot(ei, w1_ref[...], preferred_element_type=jnp.float32)
        + b1_ref[...],
        0.0,
    )
    out_ref[...] = (
        jnp.dot(z, w2_ref[...], preferred_element_type=jnp.float32)
        + b2_ref[...]
    )


def _tc_cls(o_s, o_d, o_e, cW1, cb1, cW2, cb2):
    blk1 = jnp.pad(cW1[0:H_F], ((0, 16 - H_F), (0, 8 - H_F)))
    blk2 = jnp.pad(cW1[H_F:2 * H_F], ((0, 16 - H_F), (0, 8 - H_F)))
    blk3 = jnp.pad(cW1[2 * H_F:], ((0, 0), (0, 8 - H_F)))
    cW1p = jnp.concatenate([blk1, blk2, blk3], axis=0)
    cb1p = jnp.pad(cb1, (0, 8 - H_F)).reshape(1, 8)
    cW2p = jnp.pad(cW2, ((0, 8 - H_F), (0, 8 - OUT_F)))
    cb2p = jnp.pad(cb2, (0, 8 - OUT_F)).reshape(1, 8)
    tile = 1024
    grid = KP // tile
    return pl.pallas_call(
        _tc_cls_body,
        grid=(grid,),
        in_specs=[
            pl.BlockSpec((tile, 16), lambda i: (i, 0)),
            pl.BlockSpec((tile, 16), lambda i: (i, 0)),
            pl.BlockSpec((tile, DE_F), lambda i: (i, 0)),
            pl.BlockSpec((32 + DE_F, 8), lambda i: (0, 0)),
            pl.BlockSpec((1, 8), lambda i: (0, 0)),
            pl.BlockSpec((8, 8), lambda i: (0, 0)),
            pl.BlockSpec((1, 8), lambda i: (0, 0)),
        ],
        out_specs=pl.BlockSpec((tile, 8), lambda i: (i, 0)),
        out_shape=jax.ShapeDtypeStruct((KP, 8), jnp.float32),
    )(o_s, o_d, o_e, cW1p, cb1p, cW2p, cb2p)


def kernel(node_feats, edge_feats, edge_index, edge_indices,
           W1, b1, W2, b2, conv_bias, cW1, cb1, cW2, cb2):
    src = edge_index[0]
    dst = edge_index[1]
    W2p = W2.reshape(MID_F, IN_F, H_F).transpose(0, 2, 1).reshape(MID_F, IN_F * H_F)
    b2p = b2.reshape(IN_F, H_F).T.reshape(IN_F * H_F)

    eidxp = jnp.concatenate(
        [edge_indices, jnp.zeros((KP - K_SEL,), jnp.int32)])
    es, ed, o_e = _sc_cls_gather_ids(eidxp, src, dst, edge_feats)

    x_src = _sc_gather_xsrc(node_feats, src)
    msgp = _tc_msg(edge_feats, x_src, W1, b1, W2p, b2p)
    zrows = jnp.zeros((NPAD, 8), jnp.float32)
    partials = _sc_scatter_msg(msgp, dst, zrows)
    h16 = _tc_finish(partials, conv_bias)

    o_s, o_d = _sc_cls_gather_h(es, ed, h16)
    out8 = _tc_cls(o_s, o_d, o_e, cW1, cb1, cW2, cb2)
    return out8[:K_SEL, :OUT_F]


# speedup vs baseline: 3.2717x; 1.1295x over previous
import functools

import jax
import jax.numpy as jnp
from jax import lax
from jax.experimental import pallas as pl
from jax.experimental.pallas import tpu as pltpu
from jax.experimental.pallas import tpu_sc as plsc

N_NODES = 10000
E_EDGES = 160000
IN_F = 128
DE_F = 16
H_F = 4
OUT_F = 2
MID_F = 256
K_SEL = 100000
KP = 102400

NC = 2
NS = 16
NW = NC * NS

EPT = E_EDGES // NW
EFULL = EPT // 128
ETAIL = EPT - EFULL * 128
NPAD = 10240
NPT = NPAD // NS

@functools.cache
def _mesh():
    return plsc.VectorSubcoreMesh(core_axis_name="c", subcore_axis_name="s")


def _wid():
    return lax.axis_index("s") * NC + lax.axis_index("c")


def _sc_gather_xsrc(table, idx):
    return pl.kernel(
        _sc_gather_xsrc_body,
        out_type=jax.ShapeDtypeStruct((E_EDGES, IN_F), jnp.float32),
        mesh=_mesh(),
        scratch_types=[
            pltpu.VMEM((128,), jnp.int32),
            pltpu.VMEM((128, IN_F), jnp.float32),
            pltpu.VMEM((ETAIL,), jnp.int32),
            pltpu.VMEM((ETAIL, IN_F), jnp.float32),
            pltpu.SemaphoreType.DMA,
        ],
    )(table, idx)


def _sc_gather_xsrc_body(table, idx, out, idx_v, rows_v, idx_t, rows_t, sem):
    base = _wid() * EPT

    def step(j, carry):
        off = base + j * 128
        pltpu.sync_copy(idx.at[pl.ds(off, 128)], idx_v)
        pltpu.async_copy(table.at[idx_v], rows_v, sem).wait()
        pltpu.sync_copy(rows_v, out.at[pl.ds(off, 128)])
        return carry

    lax.fori_loop(0, EFULL, step, 0)
    off = base + EFULL * 128
    pltpu.sync_copy(idx.at[pl.ds(off, ETAIL)], idx_t)
    pltpu.async_copy(table.at[idx_t], rows_t, sem).wait()
    pltpu.sync_copy(rows_t, out.at[pl.ds(off, ETAIL)])


def _tc_msg_body(eft_ref, xs_ref, w1_ref, b1_ref, w2p_ref, b2p_ref, out_ref):
    he = jnp.maximum(
        jax.lax.dot_general(eft_ref[...], w1_ref[...], (((0,), (0,)), ((), ())),
                            preferred_element_type=jnp.float32)
        + b1_ref[...],
        0.0,
    )
    we2 = (
        jnp.dot(he.astype(jnp.bfloat16), w2p_ref[...],
                preferred_element_type=jnp.float32)
        + b2p_ref[...]
    )
    xs = xs_ref[...]
    cols = [
        jnp.sum(xs * we2[:, o * IN_F:(o + 1) * IN_F], axis=1, keepdims=True)
        for o in range(H_F)
    ]
    ones = jnp.ones_like(cols[0])
    zeros = jnp.zeros((xs.shape[0], 8 - H_F - 1), jnp.float32)
    out_ref[...] = jnp.concatenate(cols + [ones, zeros], axis=1)


def _tc_msg(edge_feats_t, x_src, W1, b1, W2p, b2p):
    tile = 640
    grid = E_EDGES // tile
    return pl.pallas_call(
        _tc_msg_body,
        grid=(grid,),
        in_specs=[
            pl.BlockSpec((DE_F, tile), lambda i: (0, i)),
            pl.BlockSpec((tile, IN_F), lambda i: (i, 0)),
            pl.BlockSpec((DE_F, MID_F), lambda i: (0, 0)),
            pl.BlockSpec((1, MID_F), lambda i: (0, 0)),
            pl.BlockSpec((MID_F, IN_F * H_F), lambda i: (0, 0)),
            pl.BlockSpec((1, IN_F * H_F), lambda i: (0, 0)),
        ],
        out_specs=pl.BlockSpec((tile, 8), lambda i: (i, 0)),
        out_shape=jax.ShapeDtypeStruct((E_EDGES, 8), jnp.float32),
    )(edge_feats_t, x_src, W1, b1.reshape(1, -1),
      W2p.astype(jnp.bfloat16), b2p.reshape(1, -1))


def _sc_scatter_msg(msgp, dstidx, zrows):
    return pl.kernel(
        _sc_scatter_msg_body,
        out_type=jax.ShapeDtypeStruct((NC, NPAD, 8), jnp.float32),
        mesh=_mesh(),
        scratch_types=[
            pltpu.VMEM((128,), jnp.int32),
            pltpu.VMEM((128, 8), jnp.float32),
            pltpu.VMEM((ETAIL,), jnp.int32),
            pltpu.VMEM((ETAIL, 8), jnp.float32),
            pltpu.VMEM_SHARED((NPAD, 8), jnp.float32),
        ],
        compiler_params=pltpu.CompilerParams(use_tc_tiling_on_sc=False),
    )(msgp, dstidx, zrows)


def _sc_scatter_msg_body(msgp, dstidx, zrows, out, idx_v, msg_v, idx_t, msg_t, acc):
    cid = lax.axis_index("c")
    sid = lax.axis_index("s")
    base = _wid() * EPT
    nb = sid * NPT
    pltpu.sync_copy(zrows.at[pl.ds(nb, NPT)], acc.at[pl.ds(nb, NPT)])
    plsc.subcore_barrier()

    def step(j, carry):
        off = base + j * 128
        pltpu.sync_copy(dstidx.at[pl.ds(off, 128)], idx_v)
        pltpu.sync_copy(msgp.at[pl.ds(off, 128)], msg_v)
        pltpu.sync_copy(msg_v, acc.at[idx_v], add=True)
        return carry

    lax.fori_loop(0, EFULL, step, 0)
    off = base + EFULL * 128
    pltpu.sync_copy(dstidx.at[pl.ds(off, ETAIL)], idx_t)
    pltpu.sync_copy(msgp.at[pl.ds(off, ETAIL)], msg_t)
    pltpu.sync_copy(msg_t, acc.at[idx_t], add=True)
    plsc.subcore_barrier()
    pltpu.sync_copy(acc.at[pl.ds(nb, NPT)], out.at[cid, pl.ds(nb, NPT)])


def _tc_finish_body(p_ref, bias_ref, sel_ref, out_ref):
    s = p_ref[0] + p_ref[1]
    deg = jnp.sum(s * sel_ref[...], axis=1, keepdims=True)
    h = jnp.maximum(s / jnp.maximum(deg, 1.0) + bias_ref[...], 0.0)
    mask = jnp.concatenate(
        [jnp.ones((1, H_F), jnp.float32), jnp.zeros((1, 8 - H_F), jnp.float32)],
        axis=1,
    )
    h = h * mask
    out_ref[...] = jnp.concatenate([h, jnp.zeros_like(h)], axis=1)


def _tc_finish(partials, conv_bias):
    bias8 = jnp.pad(conv_bias, (0, 8 - H_F)).reshape(1, 8)
    sel = jnp.zeros((1, 8), jnp.float32).at[0, H_F].set(1.0)
    return pl.pallas_call(
        _tc_finish_body,
        grid=(1,),
        in_specs=[
            pl.BlockSpec((NC, NPAD, 8), lambda i: (0, 0, 0)),
            pl.BlockSpec((1, 8), lambda i: (0, 0)),
            pl.BlockSpec((1, 8), lambda i: (0, 0)),
        ],
        out_specs=pl.BlockSpec((NPAD, 16), lambda i: (0, 0)),
        out_shape=jax.ShapeDtypeStruct((NPAD, 16), jnp.float32),
    )(partials, bias8, sel)


_CPT = KP // 128 // NW


def _sc_cls_gather_ids(eidx, srcids, dstids, efeat):
    return pl.kernel(
        _sc_cls_gather_ids_body,
        out_type=(
            jax.ShapeDtypeStruct((KP,), jnp.int32),
            jax.ShapeDtypeStruct((KP,), jnp.int32),
            jax.ShapeDtypeStruct((KP, DE_F), jnp.float32),
        ),
        mesh=_mesh(),
        scratch_types=[
            pltpu.VMEM((128,), jnp.int32),
            pltpu.VMEM((128,), jnp.int32),
            pltpu.VMEM((128,), jnp.int32),
            pltpu.VMEM((128, DE_F), jnp.float32),
            pltpu.SemaphoreType.DMA,
        ],
        compiler_params=pltpu.CompilerParams(use_tc_tiling_on_sc=False),
    )(eidx, srcids, dstids, efeat)


def _sc_cls_gather_ids_body(eidx, srcids, dstids, efeat, o_es, o_ed, o_e,
                            eidx_v, src_v, dst_v, ef_v, sem):
    base0 = _wid() * _CPT * 128

    def step(j, carry):
        off = base0 + j * 128
        pltpu.sync_copy(eidx.at[pl.ds(off, 128)], eidx_v)
        pltpu.async_copy(srcids.at[eidx_v], src_v, sem).wait()
        pltpu.async_copy(dstids.at[eidx_v], dst_v, sem).wait()
        pltpu.async_copy(efeat.at[eidx_v], ef_v, sem).wait()
        pltpu.sync_copy(src_v, o_es.at[pl.ds(off, 128)])
        pltpu.sync_copy(dst_v, o_ed.at[pl.ds(off, 128)])
        pltpu.sync_copy(ef_v, o_e.at[pl.ds(off, 128)])
        return carry

    lax.fori_loop(0, _CPT, step, 0)


def _sc_cls_gather_h(es, ed, h16):
    return pl.kernel(
        _sc_cls_gather_h_body,
        out_type=(
            jax.ShapeDtypeStruct((KP, 16), jnp.float32),
            jax.ShapeDtypeStruct((KP, 16), jnp.float32),
        ),
        mesh=_mesh(),
        scratch_types=[
            pltpu.VMEM((128,), jnp.int32),
            pltpu.VMEM((128,), jnp.int32),
            pltpu.VMEM((128, 16), jnp.float32),
            pltpu.VMEM((128, 16), jnp.float32),
            pltpu.SemaphoreType.DMA,
        ],
        compiler_params=pltpu.CompilerParams(use_tc_tiling_on_sc=False),
    )(es, ed, h16)


def _sc_cls_gather_h_body(es, ed, h16, o_s, o_d,
                          src_v, dst_v, hs_v, hd_v, sem):
    base0 = _wid() * _CPT * 128

    def step(j, carry):
        off = base0 + j * 128
        pltpu.sync_copy(es.at[pl.ds(off, 128)], src_v)
        pltpu.sync_copy(ed.at[pl.ds(off, 128)], dst_v)
        pltpu.async_copy(h16.at[src_v], hs_v, sem).wait()
        pltpu.async_copy(h16.at[dst_v], hd_v, sem).wait()
        pltpu.sync_copy(hs_v, o_s.at[pl.ds(off, 128)])
        pltpu.sync_copy(hd_v, o_d.at[pl.ds(off, 128)])
        return carry

    lax.fori_loop(0, _CPT, step, 0)


def _tc_cls_body(a_ref, b_ref, c_ref, ws_ref, wd_ref, we_ref, b1_ref,
                 w2_ref, b2_ref, out_ref):
    z = jnp.maximum(
        jnp.dot(a_ref[...], ws_ref[...], preferred_element_type=jnp.float32)
        + jnp.dot(b_ref[...], wd_ref[...], preferred_element_type=jnp.float32)
        + jnp.dot(c_ref[...], we_ref[...], preferred_element_type=jnp.float32)
        + b1_ref[...],
        0.0,
    )
    out_ref[...] = (
        jnp.dot(z, w2_ref[...], preferred_element_type=jnp.float32)
        + b2_ref[...]
    )


def _tc_cls(o_s, o_d, o_e, cW1, cb1, cW2, cb2):
    blk1 = jnp.pad(cW1[0:H_F], ((0, 16 - H_F), (0, 8 - H_F)))
    blk2 = jnp.pad(cW1[H_F:2 * H_F], ((0, 16 - H_F), (0, 8 - H_F)))
    blk3 = jnp.pad(cW1[2 * H_F:], ((0, 0), (0, 8 - H_F)))
    eye8 = jnp.eye(8, dtype=jnp.float32)
    Wts = jnp.kron(eye8, blk1)
    Wtd = jnp.kron(eye8, blk2)
    Wte = jnp.kron(eye8, blk3)
    cb1t = jnp.tile(jnp.pad(cb1, (0, 8 - H_F)), 8).reshape(1, 64)
    cW2pp = jnp.pad(cW2, ((0, 8 - H_F), (0, 16 - OUT_F)))
    V2 = jnp.kron(eye8, cW2pp)
    cb2t = jnp.tile(jnp.pad(cb2, (0, 16 - OUT_F)), 8).reshape(1, 128)
    rows = 128
    grid = KP // (8 * rows)
    ps = o_s.reshape(KP // 8, 128)
    pd = o_d.reshape(KP // 8, 128)
    pe = o_e.reshape(KP // 8, 128)
    out_pk = pl.pallas_call(
        _tc_cls_body,
        grid=(grid,),
        in_specs=[
            pl.BlockSpec((rows, 128), lambda i: (i, 0)),
            pl.BlockSpec((rows, 128), lambda i: (i, 0)),
            pl.BlockSpec((rows, 128), lambda i: (i, 0)),
            pl.BlockSpec((128, 64), lambda i: (0, 0)),
            pl.BlockSpec((128, 64), lambda i: (0, 0)),
            pl.BlockSpec((128, 64), lambda i: (0, 0)),
            pl.BlockSpec((1, 64), lambda i: (0, 0)),
            pl.BlockSpec((64, 128), lambda i: (0, 0)),
            pl.BlockSpec((1, 128), lambda i: (0, 0)),
        ],
        out_specs=pl.BlockSpec((rows, 128), lambda i: (i, 0)),
        out_shape=jax.ShapeDtypeStruct((KP // 8, 128), jnp.float32),
    )(ps, pd, pe, Wts, Wtd, Wte, cb1t, V2, cb2t)
    return out_pk.reshape(KP, 16)


def kernel(node_feats, edge_feats, edge_index, edge_indices,
           W1, b1, W2, b2, conv_bias, cW1, cb1, cW2, cb2):
    src = edge_index[0]
    dst = edge_index[1]
    W2p = W2.reshape(MID_F, IN_F, H_F).transpose(0, 2, 1).reshape(MID_F, IN_F * H_F)
    b2p = b2.reshape(IN_F, H_F).T.reshape(IN_F * H_F)

    eidxp = jnp.concatenate(
        [edge_indices, jnp.zeros((KP - K_SEL,), jnp.int32)])
    es, ed, o_e = _sc_cls_gather_ids(eidxp, src, dst, edge_feats)

    x_src = _sc_gather_xsrc(node_feats, src)
    msgp = _tc_msg(edge_feats.T, x_src, W1, b1, W2p, b2p)
    zrows = jnp.zeros((NPAD, 8), jnp.float32)
    partials = _sc_scatter_msg(msgp, dst, zrows)
    h16 = _tc_finish(partials, conv_bias)

    o_s, o_d = _sc_cls_gather_h(es, ed, h16)
    out16 = _tc_cls(o_s, o_d, o_e, cW1, cb1, cW2, cb2)
    return out16[:K_SEL, :OUT_F]


# speedup vs baseline: 3.8217x; 1.1681x over previous
import functools

import jax
import jax.numpy as jnp
from jax import lax
from jax.experimental import pallas as pl
from jax.experimental.pallas import tpu as pltpu
from jax.experimental.pallas import tpu_sc as plsc

N_NODES = 10000
E_EDGES = 160000
IN_F = 128
DE_F = 16
H_F = 4
OUT_F = 2
MID_F = 256
K_SEL = 100000
KP = 102400

NC = 2
NS = 16
NW = NC * NS

EPT = E_EDGES // NW
EFULL = EPT // 128
ETAIL = EPT - EFULL * 128
NPAD = 10240
NPT = NPAD // NS

@functools.cache
def _mesh():
    return plsc.VectorSubcoreMesh(core_axis_name="c", subcore_axis_name="s")


def _wid():
    return lax.axis_index("s") * NC + lax.axis_index("c")


def _sc_gather_xsrc(table, idx):
    return pl.kernel(
        _sc_gather_xsrc_body,
        out_type=jax.ShapeDtypeStruct((E_EDGES, IN_F), jnp.float32),
        mesh=_mesh(),
        scratch_types=[
            pltpu.VMEM((128,), jnp.int32),
            pltpu.VMEM((128, IN_F), jnp.float32),
            pltpu.VMEM((ETAIL,), jnp.int32),
            pltpu.VMEM((ETAIL, IN_F), jnp.float32),
            pltpu.SemaphoreType.DMA,
        ],
    )(table, idx)


def _sc_gather_xsrc_body(table, idx, out, idx_v, rows_v, idx_t, rows_t, sem):
    base = _wid() * EPT

    def step(j, carry):
        off = base + j * 128
        pltpu.sync_copy(idx.at[pl.ds(off, 128)], idx_v)
        pltpu.async_copy(table.at[idx_v], rows_v, sem).wait()
        pltpu.sync_copy(rows_v, out.at[pl.ds(off, 128)])
        return carry

    lax.fori_loop(0, EFULL, step, 0)
    off = base + EFULL * 128
    pltpu.sync_copy(idx.at[pl.ds(off, ETAIL)], idx_t)
    pltpu.async_copy(table.at[idx_t], rows_t, sem).wait()
    pltpu.sync_copy(rows_t, out.at[pl.ds(off, ETAIL)])


def _tc_msg_body(eft_ref, xs_ref, w1_ref, b1_ref, w2p_ref, b2p_ref, out_ref):
    he = jnp.maximum(
        jax.lax.dot_general(eft_ref[...].astype(jnp.bfloat16), w1_ref[...],
                            (((0,), (0,)), ((), ())),
                            preferred_element_type=jnp.float32)
        + b1_ref[...],
        0.0,
    )
    we2 = (
        jnp.dot(he.astype(jnp.bfloat16), w2p_ref[...],
                preferred_element_type=jnp.float32)
        + b2p_ref[...]
    )
    xs = xs_ref[...]
    cols = [
        jnp.sum(xs * we2[:, o * IN_F:(o + 1) * IN_F], axis=1, keepdims=True)
        for o in range(H_F)
    ]
    ones = jnp.ones_like(cols[0])
    zeros = jnp.zeros((xs.shape[0], 8 - H_F - 1), jnp.float32)
    out_ref[...] = jnp.concatenate(cols + [ones, zeros], axis=1)


def _tc_msg(edge_feats_t, x_src, W1, b1, W2p, b2p):
    tile = 1280
    grid = E_EDGES // tile
    return pl.pallas_call(
        _tc_msg_body,
        grid=(grid,),
        in_specs=[
            pl.BlockSpec((DE_F, tile), lambda i: (0, i)),
            pl.BlockSpec((tile, IN_F), lambda i: (i, 0)),
            pl.BlockSpec((DE_F, MID_F), lambda i: (0, 0)),
            pl.BlockSpec((1, MID_F), lambda i: (0, 0)),
            pl.BlockSpec((MID_F, IN_F * H_F), lambda i: (0, 0)),
            pl.BlockSpec((1, IN_F * H_F), lambda i: (0, 0)),
        ],
        out_specs=pl.BlockSpec((tile, 8), lambda i: (i, 0)),
        out_shape=jax.ShapeDtypeStruct((E_EDGES, 8), jnp.float32),
    )(edge_feats_t, x_src, W1.astype(jnp.bfloat16), b1.reshape(1, -1),
      W2p.astype(jnp.bfloat16), b2p.reshape(1, -1))


def _sc_scatter_msg(msgp, dstidx, zrows):
    return pl.kernel(
        _sc_scatter_msg_body,
        out_type=jax.ShapeDtypeStruct((NC, NPAD, 8), jnp.float32),
        mesh=_mesh(),
        scratch_types=[
            pltpu.VMEM((128,), jnp.int32),
            pltpu.VMEM((128, 8), jnp.float32),
            pltpu.VMEM((ETAIL,), jnp.int32),
            pltpu.VMEM((ETAIL, 8), jnp.float32),
            pltpu.VMEM_SHARED((NPAD, 8), jnp.float32),
        ],
        compiler_params=pltpu.CompilerParams(use_tc_tiling_on_sc=False),
    )(msgp, dstidx, zrows)


def _sc_scatter_msg_body(msgp, dstidx, zrows, out, idx_v, msg_v, idx_t, msg_t, acc):
    cid = lax.axis_index("c")
    sid = lax.axis_index("s")
    base = _wid() * EPT
    nb = sid * NPT
    pltpu.sync_copy(zrows.at[pl.ds(nb, NPT)], acc.at[pl.ds(nb, NPT)])
    plsc.subcore_barrier()

    def step(j, carry):
        off = base + j * 128
        pltpu.sync_copy(dstidx.at[pl.ds(off, 128)], idx_v)
        pltpu.sync_copy(msgp.at[pl.ds(off, 128)], msg_v)
        pltpu.sync_copy(msg_v, acc.at[idx_v], add=True)
        return carry

    lax.fori_loop(0, EFULL, step, 0)
    off = base + EFULL * 128
    pltpu.sync_copy(dstidx.at[pl.ds(off, ETAIL)], idx_t)
    pltpu.sync_copy(msgp.at[pl.ds(off, ETAIL)], msg_t)
    pltpu.sync_copy(msg_t, acc.at[idx_t], add=True)
    plsc.subcore_barrier()
    pltpu.sync_copy(acc.at[pl.ds(nb, NPT)], out.at[cid, pl.ds(nb, NPT)])


def _tc_finish_body(p_ref, bias_ref, sel_ref, out_ref):
    s = p_ref[0] + p_ref[1]
    deg = jnp.sum(s * sel_ref[...], axis=1, keepdims=True)
    h = jnp.maximum(s / jnp.maximum(deg, 1.0) + bias_ref[...], 0.0)
    mask = jnp.concatenate(
        [jnp.ones((1, H_F), jnp.float32), jnp.zeros((1, 8 - H_F), jnp.float32)],
        axis=1,
    )
    h = h * mask
    out_ref[...] = jnp.concatenate([h, jnp.zeros_like(h)], axis=1)


def _tc_finish(partials, conv_bias):
    bias8 = jnp.pad(conv_bias, (0, 8 - H_F)).reshape(1, 8)
    sel = jnp.zeros((1, 8), jnp.float32).at[0, H_F].set(1.0)
    return pl.pallas_call(
        _tc_finish_body,
        grid=(1,),
        in_specs=[
            pl.BlockSpec((NC, NPAD, 8), lambda i: (0, 0, 0)),
            pl.BlockSpec((1, 8), lambda i: (0, 0)),
            pl.BlockSpec((1, 8), lambda i: (0, 0)),
        ],
        out_specs=pl.BlockSpec((NPAD, 16), lambda i: (0, 0)),
        out_shape=jax.ShapeDtypeStruct((NPAD, 16), jnp.float32),
    )(partials, bias8, sel)


_CPT = KP // 128 // NW


def _sc_cls_gather_ids(eidx, srcids, dstids, efeat):
    return pl.kernel(
        _sc_cls_gather_ids_body,
        out_type=(
            jax.ShapeDtypeStruct((KP,), jnp.int32),
            jax.ShapeDtypeStruct((KP,), jnp.int32),
            jax.ShapeDtypeStruct((KP, DE_F), jnp.float32),
        ),
        mesh=_mesh(),
        scratch_types=[
            pltpu.VMEM((128,), jnp.int32),
            pltpu.VMEM((128,), jnp.int32),
            pltpu.VMEM((128,), jnp.int32),
            pltpu.VMEM((128, DE_F), jnp.float32),
            pltpu.SemaphoreType.DMA,
        ],
        compiler_params=pltpu.CompilerParams(use_tc_tiling_on_sc=False),
    )(eidx, srcids, dstids, efeat)


def _sc_cls_gather_ids_body(eidx, srcids, dstids, efeat, o_es, o_ed, o_e,
                            eidx_v, src_v, dst_v, ef_v, sem):
    base0 = _wid() * _CPT * 128

    def step(j, carry):
        off = base0 + j * 128
        pltpu.sync_copy(eidx.at[pl.ds(off, 128)], eidx_v)
        pltpu.async_copy(srcids.at[eidx_v], src_v, sem).wait()
        pltpu.async_copy(dstids.at[eidx_v], dst_v, sem).wait()
        pltpu.async_copy(efeat.at[eidx_v], ef_v, sem).wait()
        pltpu.sync_copy(src_v, o_es.at[pl.ds(off, 128)])
        pltpu.sync_copy(dst_v, o_ed.at[pl.ds(off, 128)])
        pltpu.sync_copy(ef_v, o_e.at[pl.ds(off, 128)])
        return carry

    lax.fori_loop(0, _CPT, step, 0)


def _sc_cls_gather_h(es, ed, h16):
    return pl.kernel(
        _sc_cls_gather_h_body,
        out_type=(
            jax.ShapeDtypeStruct((KP, 16), jnp.float32),
            jax.ShapeDtypeStruct((KP, 16), jnp.float32),
        ),
        mesh=_mesh(),
        scratch_types=[
            pltpu.VMEM((128,), jnp.int32),
            pltpu.VMEM((128,), jnp.int32),
            pltpu.VMEM((128, 16), jnp.float32),
            pltpu.VMEM((128, 16), jnp.float32),
            pltpu.SemaphoreType.DMA,
        ],
        compiler_params=pltpu.CompilerParams(use_tc_tiling_on_sc=False),
    )(es, ed, h16)


def _sc_cls_gather_h_body(es, ed, h16, o_s, o_d,
                          src_v, dst_v, hs_v, hd_v, sem):
    base0 = _wid() * _CPT * 128

    def step(j, carry):
        off = base0 + j * 128
        pltpu.sync_copy(es.at[pl.ds(off, 128)], src_v)
        pltpu.sync_copy(ed.at[pl.ds(off, 128)], dst_v)
        pltpu.async_copy(h16.at[src_v], hs_v, sem).wait()
        pltpu.async_copy(h16.at[dst_v], hd_v, sem).wait()
        pltpu.sync_copy(hs_v, o_s.at[pl.ds(off, 128)])
        pltpu.sync_copy(hd_v, o_d.at[pl.ds(off, 128)])
        return carry

    lax.fori_loop(0, _CPT, step, 0)


def _tc_cls_body(a_ref, b_ref, c_ref, ws_ref, wd_ref, we_ref, b1_ref,
                 w2_ref, b2_ref, out_ref):
    z = jnp.maximum(
        jnp.dot(a_ref[...], ws_ref[...], preferred_element_type=jnp.float32)
        + jnp.dot(b_ref[...], wd_ref[...], preferred_element_type=jnp.float32)
        + jnp.dot(c_ref[...], we_ref[...], preferred_element_type=jnp.float32)
        + b1_ref[...],
        0.0,
    )
    out_ref[...] = (
        jnp.dot(z, w2_ref[...], preferred_element_type=jnp.float32)
        + b2_ref[...]
    )


def _tc_cls(o_s, o_d, o_e, cW1, cb1, cW2, cb2):
    blk1 = jnp.pad(cW1[0:H_F], ((0, 16 - H_F), (0, 8 - H_F)))
    blk2 = jnp.pad(cW1[H_F:2 * H_F], ((0, 16 - H_F), (0, 8 - H_F)))
    blk3 = jnp.pad(cW1[2 * H_F:], ((0, 0), (0, 8 - H_F)))
    eye8 = jnp.eye(8, dtype=jnp.float32)
    Wts = jnp.kron(eye8, blk1)
    Wtd = jnp.kron(eye8, blk2)
    Wte = jnp.kron(eye8, blk3)
    cb1t = jnp.tile(jnp.pad(cb1, (0, 8 - H_F)), 8).reshape(1, 64)
    cW2pp = jnp.pad(cW2, ((0, 8 - H_F), (0, 16 - OUT_F)))
    V2 = jnp.kron(eye8, cW2pp)
    cb2t = jnp.tile(jnp.pad(cb2, (0, 16 - OUT_F)), 8).reshape(1, 128)
    rows = 512
    grid = KP // (8 * rows)
    ps = o_s.reshape(KP // 8, 128)
    pd = o_d.reshape(KP // 8, 128)
    pe = o_e.reshape(KP // 8, 128)
    out_pk = pl.pallas_call(
        _tc_cls_body,
        grid=(grid,),
        in_specs=[
            pl.BlockSpec((rows, 128), lambda i: (i, 0)),
            pl.BlockSpec((rows, 128), lambda i: (i, 0)),
            pl.BlockSpec((rows, 128), lambda i: (i, 0)),
            pl.BlockSpec((128, 64), lambda i: (0, 0)),
            pl.BlockSpec((128, 64), lambda i: (0, 0)),
            pl.BlockSpec((128, 64), lambda i: (0, 0)),
            pl.BlockSpec((1, 64), lambda i: (0, 0)),
            pl.BlockSpec((64, 128), lambda i: (0, 0)),
            pl.BlockSpec((1, 128), lambda i: (0, 0)),
        ],
        out_specs=pl.BlockSpec((rows, 128), lambda i: (i, 0)),
        out_shape=jax.ShapeDtypeStruct((KP // 8, 128), jnp.float32),
    )(ps, pd, pe, Wts, Wtd, Wte, cb1t, V2, cb2t)
    return out_pk.reshape(KP, 16)


def kernel(node_feats, edge_feats, edge_index, edge_indices,
           W1, b1, W2, b2, conv_bias, cW1, cb1, cW2, cb2):
    src = edge_index[0]
    dst = edge_index[1]
    W2p = W2.reshape(MID_F, IN_F, H_F).transpose(0, 2, 1).reshape(MID_F, IN_F * H_F)
    b2p = b2.reshape(IN_F, H_F).T.reshape(IN_F * H_F)

    x_src = _sc_gather_xsrc(node_feats, src)

    eidxp = jnp.concatenate(
        [edge_indices, jnp.zeros((KP - K_SEL,), jnp.int32)])
    es, ed, o_e = _sc_cls_gather_ids(eidxp, src, dst, edge_feats)

    msgp = _tc_msg(edge_feats.T, x_src, W1, b1, W2p, b2p)
    zrows = jnp.zeros((NPAD, 8), jnp.float32)
    partials = _sc_scatter_msg(msgp, dst, zrows)
    h16 = _tc_finish(partials, conv_bias)

    o_s, o_d = _sc_cls_gather_h(es, ed, h16)
    out16 = _tc_cls(o_s, o_d, o_e, cW1, cb1, cW2, cb2)
    return out16[:K_SEL, :OUT_F]


# speedup vs baseline: 4.5117x; 1.1805x over previous
import functools

import jax
import jax.numpy as jnp
from jax import lax
from jax.experimental import pallas as pl
from jax.experimental.pallas import tpu as pltpu
from jax.experimental.pallas import tpu_sc as plsc

N_NODES = 10000
E_EDGES = 160000
IN_F = 128
DE_F = 16
H_F = 4
OUT_F = 2
MID_F = 256
K_SEL = 100000
KP = 102400

NC = 2
NS = 16
NW = NC * NS

EPT = E_EDGES // NW
EFULL = EPT // 128
ETAIL = EPT - EFULL * 128
NPAD = 10240
NPT = NPAD // NS

@functools.cache
def _mesh():
    return plsc.VectorSubcoreMesh(core_axis_name="c", subcore_axis_name="s")


def _wid():
    return lax.axis_index("s") * NC + lax.axis_index("c")


def _sc_gather_xsrc(table, idx):
    return pl.kernel(
        _sc_gather_xsrc_body,
        out_type=jax.ShapeDtypeStruct((E_EDGES, IN_F), jnp.float32),
        mesh=_mesh(),
        scratch_types=[
            pltpu.VMEM((EPT,), jnp.int32),
            pltpu.VMEM((128, IN_F), jnp.float32),
            pltpu.VMEM((128, IN_F), jnp.float32),
            pltpu.VMEM((ETAIL,), jnp.int32),
            pltpu.VMEM((ETAIL, IN_F), jnp.float32),
            pltpu.SemaphoreType.DMA,
            pltpu.SemaphoreType.DMA,
        ],
    )(table, idx)


def _sc_gather_xsrc_body(table, idx, out, idx_all, buf0, buf1, idx_t, rows_t,
                         sem0, sem1):
    base = _wid() * EPT
    pltpu.sync_copy(idx.at[pl.ds(base, EPT)], idx_all)
    pltpu.async_copy(table.at[idx_all.at[pl.ds(0, 128)]], buf0, sem0)

    def step(k, carry):
        j0 = 2 * k
        pltpu.async_copy(
            table.at[idx_all.at[pl.ds((j0 + 1) * 128, 128)]], buf1, sem1)
        pltpu.make_async_copy(table.at[idx_all.at[pl.ds(0, 128)]],
                              buf0, sem0).wait()
        pltpu.sync_copy(buf0, out.at[pl.ds(base + j0 * 128, 128)])

        @pl.when(j0 + 2 < EFULL)
        def _():
            pltpu.async_copy(
                table.at[idx_all.at[pl.ds((j0 + 2) * 128, 128)]], buf0, sem0)

        pltpu.make_async_copy(table.at[idx_all.at[pl.ds(0, 128)]],
                              buf1, sem1).wait()
        pltpu.sync_copy(buf1, out.at[pl.ds(base + (j0 + 1) * 128, 128)])
        return carry

    lax.fori_loop(0, EFULL // 2, step, 0)
    j0 = EFULL - 1
    pltpu.make_async_copy(table.at[idx_all.at[pl.ds(0, 128)]],
                          buf0, sem0).wait()
    pltpu.sync_copy(buf0, out.at[pl.ds(base + j0 * 128, 128)])
    off = base + EFULL * 128
    pltpu.sync_copy(idx.at[pl.ds(off, ETAIL)], idx_t)
    pltpu.async_copy(table.at[idx_t], rows_t, sem0).wait()
    pltpu.sync_copy(rows_t, out.at[pl.ds(off, ETAIL)])


def _tc_msg_body(eft_ref, xs_ref, w1_ref, b1_ref, w2p_ref, b2p_ref, out_ref):
    he = jnp.maximum(
        jax.lax.dot_general(eft_ref[...].astype(jnp.bfloat16), w1_ref[...],
                            (((0,), (0,)), ((), ())),
                            preferred_element_type=jnp.float32)
        + b1_ref[...],
        0.0,
    )
    we2 = (
        jnp.dot(he.astype(jnp.bfloat16), w2p_ref[...],
                preferred_element_type=jnp.float32)
        + b2p_ref[...]
    )
    xs = xs_ref[...]
    cols = [
        jnp.sum(xs * we2[:, o * IN_F:(o + 1) * IN_F], axis=1, keepdims=True)
        for o in range(H_F)
    ]
    ones = jnp.ones_like(cols[0])
    zeros = jnp.zeros((xs.shape[0], 8 - H_F - 1), jnp.float32)
    out_ref[...] = jnp.concatenate(cols + [ones, zeros], axis=1)


def _tc_msg(edge_feats_t, x_src, W1, b1, W2p, b2p):
    tile = 1280
    grid = E_EDGES // tile
    return pl.pallas_call(
        _tc_msg_body,
        grid=(grid,),
        in_specs=[
            pl.BlockSpec((DE_F, tile), lambda i: (0, i)),
            pl.BlockSpec((tile, IN_F), lambda i: (i, 0)),
            pl.BlockSpec((DE_F, MID_F), lambda i: (0, 0)),
            pl.BlockSpec((1, MID_F), lambda i: (0, 0)),
            pl.BlockSpec((MID_F, IN_F * H_F), lambda i: (0, 0)),
            pl.BlockSpec((1, IN_F * H_F), lambda i: (0, 0)),
        ],
        out_specs=pl.BlockSpec((tile, 8), lambda i: (i, 0)),
        out_shape=jax.ShapeDtypeStruct((E_EDGES, 8), jnp.float32),
    )(edge_feats_t, x_src, W1.astype(jnp.bfloat16), b1.reshape(1, -1),
      W2p.astype(jnp.bfloat16), b2p.reshape(1, -1))


def _sc_scatter_msg(msgp, dstidx, zrows):
    return pl.kernel(
        _sc_scatter_msg_body,
        out_type=jax.ShapeDtypeStruct((NC, NPAD, 8), jnp.float32),
        mesh=_mesh(),
        scratch_types=[
            pltpu.VMEM((128,), jnp.int32),
            pltpu.VMEM((128, 8), jnp.float32),
            pltpu.VMEM((128,), jnp.int32),
            pltpu.VMEM((128, 8), jnp.float32),
            pltpu.VMEM((ETAIL,), jnp.int32),
            pltpu.VMEM((ETAIL, 8), jnp.float32),
            pltpu.VMEM_SHARED((NPAD, 8), jnp.float32),
            pltpu.SemaphoreType.DMA,
            pltpu.SemaphoreType.DMA,
        ],
        compiler_params=pltpu.CompilerParams(use_tc_tiling_on_sc=False),
    )(msgp, dstidx, zrows)


def _sc_scatter_msg_body(msgp, dstidx, zrows, out, idx0, msg0, idx1, msg1,
                         idx_t, msg_t, acc, sem0, sem1):
    cid = lax.axis_index("c")
    sid = lax.axis_index("s")
    base = _wid() * EPT
    nb = sid * NPT
    pltpu.sync_copy(zrows.at[pl.ds(nb, NPT)], acc.at[pl.ds(nb, NPT)])
    plsc.subcore_barrier()
    pltpu.async_copy(dstidx.at[pl.ds(base, 128)], idx0, sem0)
    pltpu.async_copy(msgp.at[pl.ds(base, 128)], msg0, sem0)

    def step(k, carry):
        j0 = 2 * k
        off1 = base + (j0 + 1) * 128
        pltpu.async_copy(dstidx.at[pl.ds(off1, 128)], idx1, sem1)
        pltpu.async_copy(msgp.at[pl.ds(off1, 128)], msg1, sem1)
        pltpu.make_async_copy(dstidx.at[pl.ds(base, 128)], idx0, sem0).wait()
        pltpu.make_async_copy(msgp.at[pl.ds(base, 128)], msg0, sem0).wait()
        pltpu.sync_copy(msg0, acc.at[idx0], add=True)

        @pl.when(j0 + 2 < EFULL)
        def _():
            off2 = base + (j0 + 2) * 128
            pltpu.async_copy(dstidx.at[pl.ds(off2, 128)], idx0, sem0)
            pltpu.async_copy(msgp.at[pl.ds(off2, 128)], msg0, sem0)

        pltpu.make_async_copy(dstidx.at[pl.ds(base, 128)], idx1, sem1).wait()
        pltpu.make_async_copy(msgp.at[pl.ds(base, 128)], msg1, sem1).wait()
        pltpu.sync_copy(msg1, acc.at[idx1], add=True)
        return carry

    lax.fori_loop(0, EFULL // 2, step, 0)
    pltpu.make_async_copy(dstidx.at[pl.ds(base, 128)], idx0, sem0).wait()
    pltpu.make_async_copy(msgp.at[pl.ds(base, 128)], msg0, sem0).wait()
    pltpu.sync_copy(msg0, acc.at[idx0], add=True)
    off = base + EFULL * 128
    pltpu.sync_copy(dstidx.at[pl.ds(off, ETAIL)], idx_t)
    pltpu.sync_copy(msgp.at[pl.ds(off, ETAIL)], msg_t)
    pltpu.sync_copy(msg_t, acc.at[idx_t], add=True)
    plsc.subcore_barrier()
    pltpu.sync_copy(acc.at[pl.ds(nb, NPT)], out.at[cid, pl.ds(nb, NPT)])


def _tc_finish_body(p_ref, bias_ref, sel_ref, out_ref):
    s = p_ref[0] + p_ref[1]
    deg = jnp.sum(s * sel_ref[...], axis=1, keepdims=True)
    h = jnp.maximum(s / jnp.maximum(deg, 1.0) + bias_ref[...], 0.0)
    mask = jnp.concatenate(
        [jnp.ones((1, H_F), jnp.float32), jnp.zeros((1, 8 - H_F), jnp.float32)],
        axis=1,
    )
    h = h * mask
    out_ref[...] = jnp.concatenate([h, jnp.zeros_like(h)], axis=1)


def _tc_finish(partials, conv_bias):
    bias8 = jnp.pad(conv_bias, (0, 8 - H_F)).reshape(1, 8)
    sel = jnp.zeros((1, 8), jnp.float32).at[0, H_F].set(1.0)
    return pl.pallas_call(
        _tc_finish_body,
        grid=(1,),
        in_specs=[
            pl.BlockSpec((NC, NPAD, 8), lambda i: (0, 0, 0)),
            pl.BlockSpec((1, 8), lambda i: (0, 0)),
            pl.BlockSpec((1, 8), lambda i: (0, 0)),
        ],
        out_specs=pl.BlockSpec((NPAD, 16), lambda i: (0, 0)),
        out_shape=jax.ShapeDtypeStruct((NPAD, 16), jnp.float32),
    )(partials, bias8, sel)


_CPT = KP // 128 // NW


def _sc_cls_gather_ids(eidx, srcids, dstids, efeat):
    return pl.kernel(
        _sc_cls_gather_ids_body,
        out_type=(
            jax.ShapeDtypeStruct((KP,), jnp.int32),
            jax.ShapeDtypeStruct((KP,), jnp.int32),
            jax.ShapeDtypeStruct((KP, DE_F), jnp.float32),
        ),
        mesh=_mesh(),
        scratch_types=[
            pltpu.VMEM((128,), jnp.int32),
            pltpu.VMEM((128,), jnp.int32),
            pltpu.VMEM((128,), jnp.int32),
            pltpu.VMEM((128, DE_F), jnp.float32),
            pltpu.SemaphoreType.DMA,
        ],
        compiler_params=pltpu.CompilerParams(use_tc_tiling_on_sc=False),
    )(eidx, srcids, dstids, efeat)


def _sc_cls_gather_ids_body(eidx, srcids, dstids, efeat, o_es, o_ed, o_e,
                            eidx_v, src_v, dst_v, ef_v, sem):
    base0 = _wid() * _CPT * 128

    def step(j, carry):
        off = base0 + j * 128
        pltpu.sync_copy(eidx.at[pl.ds(off, 128)], eidx_v)
        pltpu.async_copy(srcids.at[eidx_v], src_v, sem).wait()
        pltpu.async_copy(dstids.at[eidx_v], dst_v, sem).wait()
        pltpu.async_copy(efeat.at[eidx_v], ef_v, sem).wait()
        pltpu.sync_copy(src_v, o_es.at[pl.ds(off, 128)])
        pltpu.sync_copy(dst_v, o_ed.at[pl.ds(off, 128)])
        pltpu.sync_copy(ef_v, o_e.at[pl.ds(off, 128)])
        return carry

    lax.fori_loop(0, _CPT, step, 0)


def _sc_cls_gather_h(es, ed, h16):
    return pl.kernel(
        _sc_cls_gather_h_body,
        out_type=(
            jax.ShapeDtypeStruct((KP, 16), jnp.float32),
            jax.ShapeDtypeStruct((KP, 16), jnp.float32),
        ),
        mesh=_mesh(),
        scratch_types=[
            pltpu.VMEM((_CPT * 128,), jnp.int32),
            pltpu.VMEM((_CPT * 128,), jnp.int32),
            pltpu.VMEM((128, 16), jnp.float32),
            pltpu.VMEM((128, 16), jnp.float32),
            pltpu.VMEM((128, 16), jnp.float32),
            pltpu.VMEM((128, 16), jnp.float32),
            pltpu.SemaphoreType.DMA,
            pltpu.SemaphoreType.DMA,
        ],
        compiler_params=pltpu.CompilerParams(use_tc_tiling_on_sc=False),
    )(es, ed, h16)


def _sc_cls_gather_h_body(es, ed, h16, o_s, o_d,
                          es_all, ed_all, hs0, hd0, hs1, hd1, sem0, sem1):
    base0 = _wid() * _CPT * 128
    pltpu.sync_copy(es.at[pl.ds(base0, _CPT * 128)], es_all)
    pltpu.sync_copy(ed.at[pl.ds(base0, _CPT * 128)], ed_all)
    pltpu.async_copy(h16.at[es_all.at[pl.ds(0, 128)]], hs0, sem0)
    pltpu.async_copy(h16.at[ed_all.at[pl.ds(0, 128)]], hd0, sem0)

    def step(k, carry):
        j0 = 2 * k
        pltpu.async_copy(h16.at[es_all.at[pl.ds((j0 + 1) * 128, 128)]],
                         hs1, sem1)
        pltpu.async_copy(h16.at[ed_all.at[pl.ds((j0 + 1) * 128, 128)]],
                         hd1, sem1)
        pltpu.make_async_copy(h16.at[es_all.at[pl.ds(0, 128)]], hs0,
                              sem0).wait()
        pltpu.make_async_copy(h16.at[ed_all.at[pl.ds(0, 128)]], hd0,
                              sem0).wait()
        pltpu.sync_copy(hs0, o_s.at[pl.ds(base0 + j0 * 128, 128)])
        pltpu.sync_copy(hd0, o_d.at[pl.ds(base0 + j0 * 128, 128)])

        @pl.when(j0 + 2 < _CPT)
        def _():
            pltpu.async_copy(h16.at[es_all.at[pl.ds((j0 + 2) * 128, 128)]],
                             hs0, sem0)
            pltpu.async_copy(h16.at[ed_all.at[pl.ds((j0 + 2) * 128, 128)]],
                             hd0, sem0)

        pltpu.make_async_copy(h16.at[es_all.at[pl.ds(0, 128)]], hs1,
                              sem1).wait()
        pltpu.make_async_copy(h16.at[ed_all.at[pl.ds(0, 128)]], hd1,
                              sem1).wait()
        pltpu.sync_copy(hs1, o_s.at[pl.ds(base0 + (j0 + 1) * 128, 128)])
        pltpu.sync_copy(hd1, o_d.at[pl.ds(base0 + (j0 + 1) * 128, 128)])
        return carry

    lax.fori_loop(0, _CPT // 2, step, 0)
    j0 = _CPT - 1
    pltpu.make_async_copy(h16.at[es_all.at[pl.ds(0, 128)]], hs0, sem0).wait()
    pltpu.make_async_copy(h16.at[ed_all.at[pl.ds(0, 128)]], hd0, sem0).wait()
    pltpu.sync_copy(hs0, o_s.at[pl.ds(base0 + j0 * 128, 128)])
    pltpu.sync_copy(hd0, o_d.at[pl.ds(base0 + j0 * 128, 128)])


def _tc_cls_body(a_ref, b_ref, c_ref, ws_ref, wd_ref, we_ref, b1_ref,
                 w2_ref, b2_ref, out_ref):
    z = jnp.maximum(
        jnp.dot(a_ref[...], ws_ref[...], preferred_element_type=jnp.float32)
        + jnp.dot(b_ref[...], wd_ref[...], preferred_element_type=jnp.float32)
        + jnp.dot(c_ref[...], we_ref[...], preferred_element_type=jnp.float32)
        + b1_ref[...],
        0.0,
    )
    out_ref[...] = (
        jnp.dot(z, w2_ref[...], preferred_element_type=jnp.float32)
        + b2_ref[...]
    )


def _tc_cls(o_s, o_d, o_e, cW1, cb1, cW2, cb2):
    blk1 = jnp.pad(cW1[0:H_F], ((0, 16 - H_F), (0, 8 - H_F)))
    blk2 = jnp.pad(cW1[H_F:2 * H_F], ((0, 16 - H_F), (0, 8 - H_F)))
    blk3 = jnp.pad(cW1[2 * H_F:], ((0, 0), (0, 8 - H_F)))
    eye8 = jnp.eye(8, dtype=jnp.float32)
    Wts = jnp.kron(eye8, blk1)
    Wtd = jnp.kron(eye8, blk2)
    Wte = jnp.kron(eye8, blk3)
    cb1t = jnp.tile(jnp.pad(cb1, (0, 8 - H_F)), 8).reshape(1, 64)
    cW2pp = jnp.pad(cW2, ((0, 8 - H_F), (0, 16 - OUT_F)))
    V2 = jnp.kron(eye8, cW2pp)
    cb2t = jnp.tile(jnp.pad(cb2, (0, 16 - OUT_F)), 8).reshape(1, 128)
    rows = 512
    grid = KP // (8 * rows)
    ps = o_s.reshape(KP // 8, 128)
    pd = o_d.reshape(KP // 8, 128)
    pe = o_e.reshape(KP // 8, 128)
    out_pk = pl.pallas_call(
        _tc_cls_body,
        grid=(grid,),
        in_specs=[
            pl.BlockSpec((rows, 128), lambda i: (i, 0)),
            pl.BlockSpec((rows, 128), lambda i: (i, 0)),
            pl.BlockSpec((rows, 128), lambda i: (i, 0)),
            pl.BlockSpec((128, 64), lambda i: (0, 0)),
            pl.BlockSpec((128, 64), lambda i: (0, 0)),
            pl.BlockSpec((128, 64), lambda i: (0, 0)),
            pl.BlockSpec((1, 64), lambda i: (0, 0)),
            pl.BlockSpec((64, 128), lambda i: (0, 0)),
            pl.BlockSpec((1, 128), lambda i: (0, 0)),
        ],
        out_specs=pl.BlockSpec((rows, 128), lambda i: (i, 0)),
        out_shape=jax.ShapeDtypeStruct((KP // 8, 128), jnp.float32),
    )(ps, pd, pe, Wts, Wtd, Wte, cb1t, V2, cb2t)
    return out_pk.reshape(KP, 16)


def kernel(node_feats, edge_feats, edge_index, edge_indices,
           W1, b1, W2, b2, conv_bias, cW1, cb1, cW2, cb2):
    src = edge_index[0]
    dst = edge_index[1]
    W2p = W2.reshape(MID_F, IN_F, H_F).transpose(0, 2, 1).reshape(MID_F, IN_F * H_F)
    b2p = b2.reshape(IN_F, H_F).T.reshape(IN_F * H_F)

    x_src = _sc_gather_xsrc(node_feats, src)

    eidxp = jnp.concatenate(
        [edge_indices, jnp.zeros((KP - K_SEL,), jnp.int32)])
    es, ed, o_e = _sc_cls_gather_ids(eidxp, src, dst, edge_feats)

    msgp = _tc_msg(edge_feats.T, x_src, W1, b1, W2p, b2p)
    zrows = jnp.zeros((NPAD, 8), jnp.float32)
    partials = _sc_scatter_msg(msgp, dst, zrows)
    h16 = _tc_finish(partials, conv_bias)

    o_s, o_d = _sc_cls_gather_h(es, ed, h16)
    out16 = _tc_cls(o_s, o_d, o_e, cW1, cb1, cW2, cb2)
    return out16[:K_SEL, :OUT_F]


# speedup vs baseline: 5.2766x; 1.1695x over previous
import functools

import jax
import jax.numpy as jnp
from jax import lax
from jax.experimental import pallas as pl
from jax.experimental.pallas import tpu as pltpu
from jax.experimental.pallas import tpu_sc as plsc

N_NODES = 10000
E_EDGES = 160000
IN_F = 128
DE_F = 16
H_F = 4
OUT_F = 2
MID_F = 256
K_SEL = 100000
KP = 102400

NC = 2
NS = 16
NW = NC * NS

EPT = E_EDGES // NW
EFULL = EPT // 128
ETAIL = EPT - EFULL * 128
NPAD = 10240
NPT = NPAD // NS

@functools.cache
def _mesh():
    return plsc.VectorSubcoreMesh(core_axis_name="c", subcore_axis_name="s")


def _wid():
    return lax.axis_index("s") * NC + lax.axis_index("c")


def _sc_gather_xsrc(table, idx):
    return pl.kernel(
        _sc_gather_xsrc_body,
        out_type=jax.ShapeDtypeStruct((E_EDGES, IN_F), jnp.float32),
        mesh=_mesh(),
        scratch_types=[
            pltpu.VMEM((EPT,), jnp.int32),
            pltpu.VMEM((128, IN_F), jnp.float32),
            pltpu.VMEM((128, IN_F), jnp.float32),
            pltpu.VMEM((ETAIL,), jnp.int32),
            pltpu.VMEM((ETAIL, IN_F), jnp.float32),
            pltpu.SemaphoreType.DMA,
            pltpu.SemaphoreType.DMA,
        ],
    )(table, idx)


def _sc_gather_xsrc_body(table, idx, out, idx_all, buf0, buf1, idx_t, rows_t,
                         sem0, sem1):
    base = _wid() * EPT
    pltpu.sync_copy(idx.at[pl.ds(base, EPT)], idx_all)
    pltpu.async_copy(table.at[idx_all.at[pl.ds(0, 128)]], buf0, sem0)

    def step(k, carry):
        j0 = 2 * k
        pltpu.async_copy(
            table.at[idx_all.at[pl.ds((j0 + 1) * 128, 128)]], buf1, sem1)
        pltpu.make_async_copy(table.at[idx_all.at[pl.ds(0, 128)]],
                              buf0, sem0).wait()
        pltpu.sync_copy(buf0, out.at[pl.ds(base + j0 * 128, 128)])

        @pl.when(j0 + 2 < EFULL)
        def _():
            pltpu.async_copy(
                table.at[idx_all.at[pl.ds((j0 + 2) * 128, 128)]], buf0, sem0)

        pltpu.make_async_copy(table.at[idx_all.at[pl.ds(0, 128)]],
                              buf1, sem1).wait()
        pltpu.sync_copy(buf1, out.at[pl.ds(base + (j0 + 1) * 128, 128)])
        return carry

    lax.fori_loop(0, EFULL // 2, step, 0)
    j0 = EFULL - 1
    pltpu.make_async_copy(table.at[idx_all.at[pl.ds(0, 128)]],
                          buf0, sem0).wait()
    pltpu.sync_copy(buf0, out.at[pl.ds(base + j0 * 128, 128)])
    off = base + EFULL * 128
    pltpu.sync_copy(idx.at[pl.ds(off, ETAIL)], idx_t)
    pltpu.async_copy(table.at[idx_t], rows_t, sem0).wait()
    pltpu.sync_copy(rows_t, out.at[pl.ds(off, ETAIL)])


def _tc_msg_body(eft_ref, xs_ref, w1_ref, b1_ref, w2p_ref, b2p_ref, out_ref):
    he = jnp.maximum(
        jax.lax.dot_general(eft_ref[...].astype(jnp.bfloat16), w1_ref[...],
                            (((0,), (0,)), ((), ())),
                            preferred_element_type=jnp.float32)
        + b1_ref[...],
        0.0,
    )
    we2 = (
        jnp.dot(he.astype(jnp.bfloat16), w2p_ref[...],
                preferred_element_type=jnp.float32)
        + b2p_ref[...]
    )
    xs = xs_ref[...]
    cols = [
        jnp.sum(xs * we2[:, o * IN_F:(o + 1) * IN_F], axis=1, keepdims=True)
        for o in range(H_F)
    ]
    ones = jnp.ones_like(cols[0])
    zeros = jnp.zeros((xs.shape[0], 8 - H_F - 1), jnp.float32)
    out_ref[...] = jnp.concatenate(cols + [ones, zeros], axis=1)


def _tc_msg(edge_feats_t, x_src, W1, b1, W2p, b2p):
    tile = 3200
    grid = E_EDGES // tile
    return pl.pallas_call(
        _tc_msg_body,
        grid=(grid,),
        in_specs=[
            pl.BlockSpec((DE_F, tile), lambda i: (0, i)),
            pl.BlockSpec((tile, IN_F), lambda i: (i, 0)),
            pl.BlockSpec((DE_F, MID_F), lambda i: (0, 0)),
            pl.BlockSpec((1, MID_F), lambda i: (0, 0)),
            pl.BlockSpec((MID_F, IN_F * H_F), lambda i: (0, 0)),
            pl.BlockSpec((1, IN_F * H_F), lambda i: (0, 0)),
        ],
        out_specs=pl.BlockSpec((tile, 8), lambda i: (i, 0)),
        out_shape=jax.ShapeDtypeStruct((E_EDGES, 8), jnp.float32),
    )(edge_feats_t, x_src, W1.astype(jnp.bfloat16), b1.reshape(1, -1),
      W2p.astype(jnp.bfloat16), b2p.reshape(1, -1))


def _sc_scatter_msg(msgp, dstidx, zrows):
    return pl.kernel(
        _sc_scatter_msg_body,
        out_type=jax.ShapeDtypeStruct((NC, NPAD, 8), jnp.float32),
        mesh=_mesh(),
        scratch_types=[
            pltpu.VMEM((128,), jnp.int32),
            pltpu.VMEM((128, 8), jnp.float32),
            pltpu.VMEM((128,), jnp.int32),
            pltpu.VMEM((128, 8), jnp.float32),
            pltpu.VMEM((ETAIL,), jnp.int32),
            pltpu.VMEM((ETAIL, 8), jnp.float32),
            pltpu.VMEM_SHARED((NPAD, 8), jnp.float32),
            pltpu.SemaphoreType.DMA,
            pltpu.SemaphoreType.DMA,
        ],
        compiler_params=pltpu.CompilerParams(use_tc_tiling_on_sc=False),
    )(msgp, dstidx, zrows)


def _sc_scatter_msg_body(msgp, dstidx, zrows, out, idx0, msg0, idx1, msg1,
                         idx_t, msg_t, acc, sem0, sem1):
    cid = lax.axis_index("c")
    sid = lax.axis_index("s")
    base = _wid() * EPT
    nb = sid * NPT
    pltpu.sync_copy(zrows.at[pl.ds(nb, NPT)], acc.at[pl.ds(nb, NPT)])
    plsc.subcore_barrier()
    pltpu.async_copy(dstidx.at[pl.ds(base, 128)], idx0, sem0)
    pltpu.async_copy(msgp.at[pl.ds(base, 128)], msg0, sem0)

    def step(k, carry):
        j0 = 2 * k
        off1 = base + (j0 + 1) * 128
        pltpu.async_copy(dstidx.at[pl.ds(off1, 128)], idx1, sem1)
        pltpu.async_copy(msgp.at[pl.ds(off1, 128)], msg1, sem1)
        pltpu.make_async_copy(dstidx.at[pl.ds(base, 128)], idx0, sem0).wait()
        pltpu.make_async_copy(msgp.at[pl.ds(base, 128)], msg0, sem0).wait()
        pltpu.sync_copy(msg0, acc.at[idx0], add=True)

        @pl.when(j0 + 2 < EFULL)
        def _():
            off2 = base + (j0 + 2) * 128
            pltpu.async_copy(dstidx.at[pl.ds(off2, 128)], idx0, sem0)
            pltpu.async_copy(msgp.at[pl.ds(off2, 128)], msg0, sem0)

        pltpu.make_async_copy(dstidx.at[pl.ds(base, 128)], idx1, sem1).wait()
        pltpu.make_async_copy(msgp.at[pl.ds(base, 128)], msg1, sem1).wait()
        pltpu.sync_copy(msg1, acc.at[idx1], add=True)
        return carry

    lax.fori_loop(0, EFULL // 2, step, 0)
    pltpu.make_async_copy(dstidx.at[pl.ds(base, 128)], idx0, sem0).wait()
    pltpu.make_async_copy(msgp.at[pl.ds(base, 128)], msg0, sem0).wait()
    pltpu.sync_copy(msg0, acc.at[idx0], add=True)
    off = base + EFULL * 128
    pltpu.sync_copy(dstidx.at[pl.ds(off, ETAIL)], idx_t)
    pltpu.sync_copy(msgp.at[pl.ds(off, ETAIL)], msg_t)
    pltpu.sync_copy(msg_t, acc.at[idx_t], add=True)
    plsc.subcore_barrier()
    pltpu.sync_copy(acc.at[pl.ds(nb, NPT)], out.at[cid, pl.ds(nb, NPT)])


def _tc_finish_body(p_ref, bias_ref, sel_ref, out_ref):
    s = p_ref[0] + p_ref[1]
    deg = jnp.sum(s * sel_ref[...], axis=1, keepdims=True)
    h = jnp.maximum(s / jnp.maximum(deg, 1.0) + bias_ref[...], 0.0)
    mask = jnp.concatenate(
        [jnp.ones((1, H_F), jnp.float32), jnp.zeros((1, 8 - H_F), jnp.float32)],
        axis=1,
    )
    h = h * mask
    out_ref[...] = jnp.concatenate([h, jnp.zeros_like(h)], axis=1)


def _tc_finish(partials, conv_bias):
    bias8 = jnp.pad(conv_bias, (0, 8 - H_F)).reshape(1, 8)
    sel = jnp.zeros((1, 8), jnp.float32).at[0, H_F].set(1.0)
    return pl.pallas_call(
        _tc_finish_body,
        grid=(1,),
        in_specs=[
            pl.BlockSpec((NC, NPAD, 8), lambda i: (0, 0, 0)),
            pl.BlockSpec((1, 8), lambda i: (0, 0)),
            pl.BlockSpec((1, 8), lambda i: (0, 0)),
        ],
        out_specs=pl.BlockSpec((NPAD, 16), lambda i: (0, 0)),
        out_shape=jax.ShapeDtypeStruct((NPAD, 16), jnp.float32),
    )(partials, bias8, sel)


_CPT = KP // 128 // NW


def _sc_cls_gather_ids(eidx, srcids, dstids):
    return pl.kernel(
        _sc_cls_gather_ids_body,
        out_type=(
            jax.ShapeDtypeStruct((KP,), jnp.int32),
            jax.ShapeDtypeStruct((KP,), jnp.int32),
        ),
        mesh=_mesh(),
        scratch_types=[
            pltpu.VMEM((_CPT * 128,), jnp.int32),
            pltpu.VMEM((128,), jnp.int32),
            pltpu.VMEM((128,), jnp.int32),
            pltpu.SemaphoreType.DMA,
        ],
        compiler_params=pltpu.CompilerParams(use_tc_tiling_on_sc=False),
    )(eidx, srcids, dstids)


def _sc_cls_gather_ids_body(eidx, srcids, dstids, o_es, o_ed,
                            eidx_all, src_v, dst_v, sem):
    base0 = _wid() * _CPT * 128
    pltpu.sync_copy(eidx.at[pl.ds(base0, _CPT * 128)], eidx_all)

    def step(j, carry):
        off = base0 + j * 128
        eslice = eidx_all.at[pl.ds(j * 128, 128)]
        pltpu.async_copy(srcids.at[eslice], src_v, sem).wait()
        pltpu.async_copy(dstids.at[eslice], dst_v, sem).wait()
        pltpu.sync_copy(src_v, o_es.at[pl.ds(off, 128)])
        pltpu.sync_copy(dst_v, o_ed.at[pl.ds(off, 128)])
        return carry

    lax.fori_loop(0, _CPT, step, 0)


def _sc_cls_gather_h(es, ed, h16, eidx, efeat):
    return pl.kernel(
        _sc_cls_gather_h_body,
        out_type=(
            jax.ShapeDtypeStruct((KP, 16), jnp.float32),
            jax.ShapeDtypeStruct((KP, 16), jnp.float32),
            jax.ShapeDtypeStruct((KP, DE_F), jnp.float32),
        ),
        mesh=_mesh(),
        scratch_types=[
            pltpu.VMEM((_CPT * 128,), jnp.int32),
            pltpu.VMEM((_CPT * 128,), jnp.int32),
            pltpu.VMEM((_CPT * 128,), jnp.int32),
            pltpu.VMEM((128, 16), jnp.float32),
            pltpu.VMEM((128, 16), jnp.float32),
            pltpu.VMEM((128, DE_F), jnp.float32),
            pltpu.VMEM((128, 16), jnp.float32),
            pltpu.VMEM((128, 16), jnp.float32),
            pltpu.VMEM((128, DE_F), jnp.float32),
            pltpu.SemaphoreType.DMA,
            pltpu.SemaphoreType.DMA,
        ],
        compiler_params=pltpu.CompilerParams(use_tc_tiling_on_sc=False),
    )(es, ed, h16, eidx, efeat)


def _sc_cls_gather_h_body(es, ed, h16, eidx, efeat, o_s, o_d, o_e,
                          es_all, ed_all, ei_all, hs0, hd0, ef0, hs1, hd1,
                          ef1, sem0, sem1):
    base0 = _wid() * _CPT * 128
    pltpu.sync_copy(es.at[pl.ds(base0, _CPT * 128)], es_all)
    pltpu.sync_copy(ed.at[pl.ds(base0, _CPT * 128)], ed_all)
    pltpu.sync_copy(eidx.at[pl.ds(base0, _CPT * 128)], ei_all)

    def fire(j, hs, hd, ef, sem):
        pltpu.async_copy(h16.at[es_all.at[pl.ds(j * 128, 128)]], hs, sem)
        pltpu.async_copy(h16.at[ed_all.at[pl.ds(j * 128, 128)]], hd, sem)
        pltpu.async_copy(efeat.at[ei_all.at[pl.ds(j * 128, 128)]], ef, sem)

    def drain_write(j, hs, hd, ef, sem):
        pltpu.make_async_copy(h16.at[es_all.at[pl.ds(0, 128)]], hs,
                              sem).wait()
        pltpu.make_async_copy(h16.at[ed_all.at[pl.ds(0, 128)]], hd,
                              sem).wait()
        pltpu.make_async_copy(efeat.at[ei_all.at[pl.ds(0, 128)]], ef,
                              sem).wait()
        off = base0 + j * 128
        pltpu.sync_copy(hs, o_s.at[pl.ds(off, 128)])
        pltpu.sync_copy(hd, o_d.at[pl.ds(off, 128)])
        pltpu.sync_copy(ef, o_e.at[pl.ds(off, 128)])

    fire(0, hs0, hd0, ef0, sem0)

    def step(k, carry):
        j0 = 2 * k
        fire(j0 + 1, hs1, hd1, ef1, sem1)
        drain_write(j0, hs0, hd0, ef0, sem0)

        @pl.when(j0 + 2 < _CPT)
        def _():
            fire(j0 + 2, hs0, hd0, ef0, sem0)

        drain_write(j0 + 1, hs1, hd1, ef1, sem1)
        return carry

    lax.fori_loop(0, _CPT // 2, step, 0)
    drain_write(_CPT - 1, hs0, hd0, ef0, sem0)


def _tc_cls_body(a_ref, b_ref, c_ref, ws_ref, wd_ref, we_ref, b1_ref,
                 w2_ref, b2_ref, out_ref):
    z = jnp.maximum(
        jnp.dot(a_ref[...], ws_ref[...], preferred_element_type=jnp.float32)
        + jnp.dot(b_ref[...], wd_ref[...], preferred_element_type=jnp.float32)
        + jnp.dot(c_ref[...], we_ref[...], preferred_element_type=jnp.float32)
        + b1_ref[...],
        0.0,
    )
    out_ref[...] = (
        jnp.dot(z, w2_ref[...], preferred_element_type=jnp.float32)
        + b2_ref[...]
    )


def _tc_cls(o_s, o_d, o_e, cW1, cb1, cW2, cb2):
    blk1 = jnp.pad(cW1[0:H_F], ((0, 16 - H_F), (0, 8 - H_F)))
    blk2 = jnp.pad(cW1[H_F:2 * H_F], ((0, 16 - H_F), (0, 8 - H_F)))
    blk3 = jnp.pad(cW1[2 * H_F:], ((0, 0), (0, 8 - H_F)))
    eye8 = jnp.eye(8, dtype=jnp.float32)
    Wts = jnp.kron(eye8, blk1)
    Wtd = jnp.kron(eye8, blk2)
    Wte = jnp.kron(eye8, blk3)
    cb1t = jnp.tile(jnp.pad(cb1, (0, 8 - H_F)), 8).reshape(1, 64)
    cW2pp = jnp.pad(cW2, ((0, 8 - H_F), (0, 16 - OUT_F)))
    V2 = jnp.kron(eye8, cW2pp)
    cb2t = jnp.tile(jnp.pad(cb2, (0, 16 - OUT_F)), 8).reshape(1, 128)
    rows = 512
    grid = KP // (8 * rows)
    ps = o_s.reshape(KP // 8, 128)
    pd = o_d.reshape(KP // 8, 128)
    pe = o_e.reshape(KP // 8, 128)
    out_pk = pl.pallas_call(
        _tc_cls_body,
        grid=(grid,),
        in_specs=[
            pl.BlockSpec((rows, 128), lambda i: (i, 0)),
            pl.BlockSpec((rows, 128), lambda i: (i, 0)),
            pl.BlockSpec((rows, 128), lambda i: (i, 0)),
            pl.BlockSpec((128, 64), lambda i: (0, 0)),
            pl.BlockSpec((128, 64), lambda i: (0, 0)),
            pl.BlockSpec((128, 64), lambda i: (0, 0)),
            pl.BlockSpec((1, 64), lambda i: (0, 0)),
            pl.BlockSpec((64, 128), lambda i: (0, 0)),
            pl.BlockSpec((1, 128), lambda i: (0, 0)),
        ],
        out_specs=pl.BlockSpec((rows, 128), lambda i: (i, 0)),
        out_shape=jax.ShapeDtypeStruct((KP // 8, 128), jnp.float32),
    )(ps, pd, pe, Wts, Wtd, Wte, cb1t, V2, cb2t)
    return out_pk.reshape(KP, 16)


def kernel(node_feats, edge_feats, edge_index, edge_indices,
           W1, b1, W2, b2, conv_bias, cW1, cb1, cW2, cb2):
    src = edge_index[0]
    dst = edge_index[1]
    W2p = W2.reshape(MID_F, IN_F, H_F).transpose(0, 2, 1).reshape(MID_F, IN_F * H_F)
    b2p = b2.reshape(IN_F, H_F).T.reshape(IN_F * H_F)

    x_src = _sc_gather_xsrc(node_feats, src)

    eidxp = jnp.concatenate(
        [edge_indices, jnp.zeros((KP - K_SEL,), jnp.int32)])
    es, ed = _sc_cls_gather_ids(eidxp, src, dst)

    msgp = _tc_msg(edge_feats.T, x_src, W1, b1, W2p, b2p)
    zrows = jnp.zeros((NPAD, 8), jnp.float32)
    partials = _sc_scatter_msg(msgp, dst, zrows)
    h16 = _tc_finish(partials, conv_bias)

    o_s, o_d, o_e = _sc_cls_gather_h(es, ed, h16, eidxp, edge_feats)
    out16 = _tc_cls(o_s, o_d, o_e, cW1, cb1, cW2, cb2)
    return out16[:K_SEL, :OUT_F]
